# baseline (speedup 1.0000x reference)
import jax
import jax.numpy as jnp
from jax import lax
from jax.experimental import pallas as pl
from jax.experimental.pallas import tpu as pltpu
from jax.experimental.pallas import tpu_sc as plsc

S = 200
GN = 4096
D = 128
N_ARGS = 11
N_COMMANDS = 7
GROUP_VOCAB = 52
VOCAB_PAD = 64
ROWS = 5

NC, NS, L = 2, 16, 16
NW = NC * NS
C = 256
CHUNKS_PER_ROW = GN // C

PH = 2
SROWS_P = S // PH
CH_P = SROWS_P * CHUNKS_PER_ROW
CPW = CH_P // NW


def _make_sc_body(p):
    def _sc_relayout(args_ref, out_ref, inbuf, outbuf, sems):
        wid = lax.axis_index("s") * NC + lax.axis_index("c")
        iota = lax.broadcasted_iota(jnp.int32, (L,), 0)
        zero = jnp.zeros((L,), jnp.float32)

        def zrow(j, _):
            for q in range(2):
                for k in range(N_ARGS, 16):
                    outbuf[q, k, pl.ds(j * L, L)] = zero
            return 0
        lax.fori_loop(0, C // L, zrow, 0)

        def in_copy(i, q):
            n = p * CH_P + i * NW + wid
            s_idx = n // CHUNKS_PER_ROW
            g0 = (n % CHUNKS_PER_ROW) * C
            return pltpu.make_async_copy(
                args_ref.at[s_idx, pl.ds(g0, C), :], inbuf.at[q], sems.at[q])

        def out_copy(i, q):
            n = p * CH_P + i * NW + wid
            s_loc = n // CHUNKS_PER_ROW - p * SROWS_P
            g0 = (n % CHUNKS_PER_ROW) * C
            return pltpu.make_async_copy(
                outbuf.at[q], out_ref.at[s_loc, :, pl.ds(g0, C)], sems.at[2 + q])

        def xpose(q):
            def body(j, _):
                rows = j * L + iota
                for k in range(N_ARGS):
                    v = plsc.load_gather(inbuf.at[q],
                                         [rows, jnp.full((L,), k, jnp.int32)])
                    outbuf[q, k, pl.ds(j * L, L)] = v
                return 0
            lax.fori_loop(0, C // L, body, 0)

        in_copy(0, 0).start()

        def pair(i, _):
            in_copy(2 * i + 1, 1).start()
            in_copy(2 * i, 0).wait()

            @pl.when(i > 0)
            def _():
                out_copy(2 * i - 2, 0).wait()
            xpose(0)
            out_copy(2 * i, 0).start()

            @pl.when(2 * i + 2 < CPW)
            def _():
                in_copy(2 * i + 2, 0).start()
            in_copy(2 * i + 1, 1).wait()

            @pl.when(i > 0)
            def _():
                out_copy(2 * i - 1, 1).wait()
            xpose(1)
            out_copy(2 * i + 1, 1).start()
            return 0

        lax.fori_loop(0, CPW // 2, pair, 0)
        out_copy(CPW - 2, 0).wait()
        out_copy(CPW - 1, 1).wait()

    return _sc_relayout


def _compute_rows(cmd_ref, grp_ref, args_ref, w1_ref, w2_ref, b_ref, pos_ref,
                  store):
    iota = lax.broadcasted_iota(jnp.int32, (VOCAB_PAD, 1), 0)
    for r in range(ROWS):
        c = cmd_ref[r]
        g = grp_ref[r]
        oh_t = (iota == c).astype(jnp.float32) + (iota == g + N_COMMANDS).astype(jnp.float32)
        acc = lax.dot_general(
            oh_t, w1_ref[...], (((0,), (0,)), ((), ())),
            preferred_element_type=jnp.float32,
        )
        acc = acc + lax.dot_general(
            args_ref[r], w2_ref[...], (((0,), (0,)), ((), ())),
            preferred_element_type=jnp.float32,
        )
        pb = pos_ref[r] + b_ref[...]
        store(r, acc + pb)


def _tc_body(cmd_ref, grp_ref, args_ref, w1_ref, w2_ref, b_ref, pos_ref, out_ref):
    _compute_rows(cmd_ref, grp_ref, args_ref, w1_ref, w2_ref, b_ref, pos_ref,
                  lambda r, v: out_ref.__setitem__(r, v))


def _make_tc_alias_body(off_rows, nsteps):
    def body(cmd_ref, grp_ref, args_ref, w1_ref, w2_ref, b_ref, pos_ref,
             prev_ref, out_ref, obuf, sems):
        s = pl.program_id(0)
        slot = s % 2

        @pl.when(s >= 2)
        def _():
            pltpu.make_async_copy(
                obuf.at[slot], out_ref.at[pl.ds(0, ROWS)], sems.at[slot]).wait()

        _compute_rows(cmd_ref, grp_ref, args_ref, w1_ref, w2_ref, b_ref, pos_ref,
                      lambda r, v: obuf.__setitem__((slot, r), v))
        pltpu.make_async_copy(
            obuf.at[slot],
            out_ref.at[pl.ds((off_rows + s * ROWS) * 1, ROWS)],
            sems.at[slot],
        ).start()

        @pl.when(s == nsteps - 1)
        def _():
            pltpu.make_async_copy(
                obuf.at[1 - slot], out_ref.at[pl.ds(0, ROWS)], sems.at[1 - slot]).wait()
            pltpu.make_async_copy(
                obuf.at[slot], out_ref.at[pl.ds(0, ROWS)], sems.at[slot]).wait()

    return body


def kernel(commands, args, groups, command_embed, W_fcn, b_fcn, group_embed, pos_embed):
    w1 = jnp.concatenate(
        [command_embed, group_embed,
         jnp.zeros((VOCAB_PAD - N_COMMANDS - GROUP_VOCAB, D), jnp.float32)], axis=0)
    w2 = jnp.concatenate([W_fcn.T, jnp.zeros((16 - N_ARGS, D), jnp.float32)], axis=0)
    b2 = b_fcn.reshape(1, D)
    cmd3 = commands.reshape(S, 1, GN).astype(jnp.int32)
    grp3 = groups.reshape(S, 1, GN).astype(jnp.int32)
    pos3 = pos_embed.reshape(-1, 1, D)

    args_c = []
    for p in range(PH):
        sc = pl.kernel(
            _make_sc_body(p),
            out_type=jax.ShapeDtypeStruct((SROWS_P, 16, GN), jnp.float32),
            mesh=plsc.VectorSubcoreMesh(core_axis_name="c", subcore_axis_name="s"),
            scratch_types=[
                pltpu.VMEM((2, C, N_ARGS), jnp.float32),
                pltpu.VMEM((2, 16, C), jnp.float32),
                pltpu.SemaphoreType.DMA((4,)),
            ],
            compiler_params=pltpu.CompilerParams(use_tc_tiling_on_sc=True,
                                                 needs_layout_passes=False),
        )
        args_c.append(sc(args))

    steps_p = SROWS_P // ROWS
    out = None
    for p in range(PH):
        off = p * steps_p
        in_specs = [
            pl.BlockSpec((ROWS, 1, GN), lambda s, off=off: (s + off, 0, 0)),
            pl.BlockSpec((ROWS, 1, GN), lambda s, off=off: (s + off, 0, 0)),
            pl.BlockSpec((ROWS, 16, GN), lambda s: (s, 0, 0)),
            pl.BlockSpec((VOCAB_PAD, D), lambda s: (0, 0)),
            pl.BlockSpec((16, D), lambda s: (0, 0)),
            pl.BlockSpec((1, D), lambda s: (0, 0)),
            pl.BlockSpec((ROWS, 1, D), lambda s, off=off: (s + off, 0, 0)),
        ]
        operands = [cmd3, grp3, args_c[p], w1, w2, b2, pos3]
        if p == 0:
            out = pl.pallas_call(
                _tc_body,
                grid=(steps_p,),
                in_specs=in_specs,
                out_specs=pl.BlockSpec((ROWS, GN, D),
                                       lambda s, off=off: (s + off, 0, 0)),
                out_shape=jax.ShapeDtypeStruct((S, GN, D), jnp.float32),
            )(*operands)
        else:
            in_specs.append(pl.BlockSpec(memory_space=pl.ANY))
            operands.append(out)
            out = pl.pallas_call(
                _make_tc_alias_body(p * SROWS_P, steps_p),
                grid=(steps_p,),
                in_specs=in_specs,
                out_specs=pl.BlockSpec(memory_space=pl.ANY),
                out_shape=jax.ShapeDtypeStruct((S, GN, D), jnp.float32),
                input_output_aliases={7: 0},
                scratch_shapes=[
                    pltpu.VMEM((2, ROWS, GN, D), jnp.float32),
                    pltpu.SemaphoreType.DMA((2,)),
                ],
            )(*operands)
    return out


# speedup vs baseline: 1.2693x; 1.2693x over previous
import jax
import jax.numpy as jnp
from jax import lax
from jax.experimental import pallas as pl

S = 200
GN = 4096
D = 128
N_COMMANDS = 7
GROUP_VOCAB = 52
VOCAB_PAD = 64
ROWS = 5


def _body(cmd_ref, grp_ref, args_ref, w1_ref, w2_ref, b_ref, pos_ref, out_ref):
    iota = lax.broadcasted_iota(jnp.int32, (VOCAB_PAD, 1), 0)
    for r in range(ROWS):
        c = cmd_ref[r]
        g = grp_ref[r]
        oh_t = (iota == c).astype(jnp.float32) + (iota == g + N_COMMANDS).astype(jnp.float32)
        acc = lax.dot_general(
            oh_t, w1_ref[...], (((0,), (0,)), ((), ())),
            preferred_element_type=jnp.float32,
        )
        acc = acc + jnp.dot(args_ref[r], w2_ref[...],
                            preferred_element_type=jnp.float32)
        pb = pos_ref[r] + b_ref[...]
        out_ref[r] = acc + pb


def kernel(commands, args, groups, command_embed, W_fcn, b_fcn, group_embed, pos_embed):
    w1 = jnp.concatenate(
        [command_embed, group_embed,
         jnp.zeros((VOCAB_PAD - N_COMMANDS - GROUP_VOCAB, D), jnp.float32)], axis=0)
    w2 = W_fcn.T
    b2 = b_fcn.reshape(1, D)
    cmd3 = commands.reshape(S, 1, GN).astype(jnp.int32)
    grp3 = groups.reshape(S, 1, GN).astype(jnp.int32)
    pos3 = pos_embed.reshape(-1, 1, D)

    grid = (S // ROWS,)
    out = pl.pallas_call(
        _body,
        grid=grid,
        in_specs=[
            pl.BlockSpec((ROWS, 1, GN), lambda s: (s, 0, 0)),
            pl.BlockSpec((ROWS, 1, GN), lambda s: (s, 0, 0)),
            pl.BlockSpec((ROWS, GN, args.shape[-1]), lambda s: (s, 0, 0)),
            pl.BlockSpec((VOCAB_PAD, D), lambda s: (0, 0)),
            pl.BlockSpec((W_fcn.shape[1], D), lambda s: (0, 0)),
            pl.BlockSpec((1, D), lambda s: (0, 0)),
            pl.BlockSpec((ROWS, 1, D), lambda s: (s, 0, 0)),
        ],
        out_specs=pl.BlockSpec((ROWS, GN, D), lambda s: (s, 0, 0)),
        out_shape=jax.ShapeDtypeStruct((S, GN, D), jnp.float32),
    )(cmd3, grp3, args, w1, w2, b2, pos3)
    return out
